# baseline (speedup 1.0000x reference)
import functools

import jax
import jax.numpy as jnp
from jax import lax
from jax.experimental import pallas as pl
from jax.experimental.pallas import tpu as pltpu
from jax.experimental.pallas import tpu_sc as plsc

NUM_CORES = 2
NUM_SUBCORES = 16
NUM_WORKERS = NUM_CORES * NUM_SUBCORES
LANES = 16
NBUF = 4
NCHUNK = 4


def kernel(query_index, knowledge, knowledge_len):
    B, S = query_index.shape
    K, T, L = knowledge.shape
    NP = L * T
    CK = K // NCHUNK

    kt = jnp.transpose(knowledge, (2, 1, 0))
    klt = lax.bitcast_convert_type(
        jnp.transpose(knowledge_len, (1, 0)), jnp.float32)
    qit = jnp.transpose(query_index.astype(jnp.int32), (1, 0))

    mesh = plsc.VectorSubcoreMesh(core_axis_name="c", subcore_axis_name="s")

    @functools.partial(
        pl.kernel,
        mesh=mesh,
        compiler_params=pltpu.CompilerParams(needs_layout_passes=False),
        out_type=[
            jax.ShapeDtypeStruct((L, S * T, B), jnp.float32),
            jax.ShapeDtypeStruct((S * T, B), jnp.float32),
        ],
        scratch_types=[
            pltpu.VMEM((K,), jnp.float32),
            pltpu.VMEM((S, B), jnp.int32),
            pltpu.VMEM((NBUF * B,), jnp.float32),
            pltpu.SemaphoreType.DMA,
            pltpu.SemaphoreType.DMA,
        ],
    )
    def gather_kernel(qit_hbm, kt_hbm, klt_hbm, ot_hbm, olt_hbm,
                      plane_v, qi_v, ob_v, sem_in, sem_out):
        w = lax.axis_index("s") * NUM_CORES + lax.axis_index("c")
        pltpu.sync_copy(qit_hbm, qi_v)

        def load_plane(src_row):
            pltpu.async_copy(src_row, plane_v, sem_in).wait()

        def gather_row_into(s, buf):
            G = 16
            for k0 in range(0, B // LANES, G):
                qvs = [qi_v[s, pl.ds((k0 + j) * LANES, LANES)]
                       for j in range(G)]
                gs = [plsc.load_gather(plane_v, [qv]) for qv in qvs]
                for j in range(G):
                    ob_v[pl.ds(buf * B + (k0 + j) * LANES, LANES)] = gs[j]

        def row_out(buf, dst_row):
            pltpu.async_copy(ob_v.at[pl.ds(buf * B, B)], dst_row, sem_out)

        def drain_one():
            pltpu.make_async_copy(ob_v.at[pl.ds(0, B)], ot_hbm.at[0, 0],
                                  sem_out).wait()

        n_f = (NP - 1 - w) // NUM_WORKERS + 1

        def f32_body(i, carry):
            p = w + i * NUM_WORKERS
            l = p // T
            t = p - l * T
            load_plane(kt_hbm.at[l, t])

            def s_body(s, r):
                buf = lax.rem(r, NBUF)

                @pl.when(r >= NBUF)
                def _():
                    drain_one()

                gather_row_into(s, buf)
                row_out(buf, ot_hbm.at[l, s * T + t])
                return r + 1

            return lax.fori_loop(0, S, s_body, carry)

        r = lax.fori_loop(0, n_f, f32_body, 0)

        @pl.when(jnp.logical_and(w >= 8, w < 8 + T))
        def _():
            t = w - 8
            load_plane(klt_hbm.at[t])

            def s_body(s, r):
                buf = lax.rem(r, NBUF)
                drain_one()
                gather_row_into(s, buf)
                row_out(buf, olt_hbm.at[s * T + t])
                return r + 1

            lax.fori_loop(0, S, s_body, r)

        for _ in range(NBUF):
            drain_one()

    ot, olt = gather_kernel(qit, kt, klt)
    batch_tensors = jnp.transpose(ot, (2, 1, 0))
    batch_len_tensors = lax.bitcast_convert_type(
        jnp.transpose(olt, (1, 0)), jnp.int32)
    return batch_tensors, batch_len_tensors


# speedup vs baseline: 1.0060x; 1.0060x over previous
import functools

import jax
import jax.numpy as jnp
from jax import lax
from jax.experimental import pallas as pl
from jax.experimental.pallas import tpu as pltpu
from jax.experimental.pallas import tpu_sc as plsc

NUM_CORES = 2
NUM_SUBCORES = 16
NUM_WORKERS = NUM_CORES * NUM_SUBCORES
LANES = 16
NBUF = 4


def kernel(query_index, knowledge, knowledge_len):
    B, S = query_index.shape
    K, T, L = knowledge.shape
    NP = L * T

    kt = jnp.transpose(knowledge, (2, 1, 0))
    klt = lax.bitcast_convert_type(
        jnp.transpose(knowledge_len, (1, 0)), jnp.float32)
    qit = jnp.transpose(query_index.astype(jnp.int32), (1, 0))

    mesh = plsc.VectorSubcoreMesh(core_axis_name="c", subcore_axis_name="s")

    @functools.partial(
        pl.kernel,
        mesh=mesh,
        compiler_params=pltpu.CompilerParams(needs_layout_passes=False),
        out_type=[
            jax.ShapeDtypeStruct((L, S * T, B), jnp.float32),
            jax.ShapeDtypeStruct((S * T, B), jnp.float32),
        ],
        scratch_types=[
            pltpu.VMEM((K,), jnp.float32),
            pltpu.VMEM((S, B), jnp.int32),
            pltpu.VMEM((NBUF * B,), jnp.float32),
            pltpu.SemaphoreType.DMA,
            pltpu.SemaphoreType.DMA,
        ],
    )
    def gather_kernel(qit_hbm, kt_hbm, klt_hbm, ot_hbm, olt_hbm,
                      plane_v, qi_v, ob_v, sem_in, sem_out):
        w = lax.axis_index("s") * NUM_CORES + lax.axis_index("c")
        pltpu.sync_copy(qit_hbm, qi_v)

        def load_plane(src_row):
            pltpu.async_copy(src_row, plane_v, sem_in).wait()

        def gather_row_into(s, buf):
            G = 8
            for k0 in range(0, B // LANES, G):
                qvs = [qi_v[s, pl.ds((k0 + j) * LANES, LANES)]
                       for j in range(G)]
                gs = [plsc.load_gather(plane_v, [qv]) for qv in qvs]
                for j in range(G):
                    ob_v[pl.ds(buf * B + (k0 + j) * LANES, LANES)] = gs[j]

        def row_out(buf, dst_row):
            pltpu.async_copy(ob_v.at[pl.ds(buf * B, B)], dst_row, sem_out)

        def drain_one():
            pltpu.make_async_copy(ob_v.at[pl.ds(0, B)], ot_hbm.at[0, 0],
                                  sem_out).wait()

        n_f = (NP - 1 - w) // NUM_WORKERS + 1

        def f32_body(i, carry):
            p = w + i * NUM_WORKERS
            l = p // T
            t = p - l * T
            load_plane(kt_hbm.at[l, t])

            def s_body(s, r):
                buf = lax.rem(r, NBUF)

                @pl.when(r >= NBUF)
                def _():
                    drain_one()

                gather_row_into(s, buf)
                row_out(buf, ot_hbm.at[l, s * T + t])
                return r + 1

            return lax.fori_loop(0, S, s_body, carry)

        r = lax.fori_loop(0, n_f, f32_body, 0)

        @pl.when(jnp.logical_and(w >= 8, w < 8 + T))
        def _():
            t = w - 8
            load_plane(klt_hbm.at[t])

            def s_body(s, r):
                buf = lax.rem(r, NBUF)
                drain_one()
                gather_row_into(s, buf)
                row_out(buf, olt_hbm.at[s * T + t])
                return r + 1

            lax.fori_loop(0, S, s_body, r)

        for _ in range(NBUF):
            drain_one()

    ot, olt = gather_kernel(qit, kt, klt)
    batch_tensors = jnp.transpose(ot, (2, 1, 0))
    batch_len_tensors = lax.bitcast_convert_type(
        jnp.transpose(olt, (1, 0)), jnp.int32)
    return batch_tensors, batch_len_tensors
